# baseline (speedup 1.0000x reference)
import dataclasses
import functools

import jax
import jax.numpy as jnp
from jax import lax
from jax.experimental import pallas as pl
from jax.experimental.pallas import tpu as pltpu
from jax.experimental.pallas import tpu_sc as plsc

_NC = 2
_NS = 16
_NW = _NC * _NS
_NBUF = 8
_LANES = 128


def _sc_gather_t(user_t, item_t, user_idx, item_idx):
    D, V = user_t.shape
    B = user_idx.shape[0]
    bpw = B // _NW

    mesh = plsc.VectorSubcoreMesh(core_axis_name="c", subcore_axis_name="s")

    cp = pltpu.CompilerParams()
    if "needs_layout_passes" in pltpu.CompilerParams.__dataclass_fields__:
        cp = dataclasses.replace(cp, needs_layout_passes=False)

    @functools.partial(
        pl.kernel,
        mesh=mesh,
        compiler_params=cp,
        out_type=(jax.ShapeDtypeStruct((D, B), jnp.float32),
                  jax.ShapeDtypeStruct((D, B), jnp.float32)),
        scratch_types=[
            pltpu.VMEM((bpw,), jnp.int32),
            pltpu.VMEM((bpw,), jnp.int32),
            pltpu.VMEM((48, D, _LANES), jnp.float32),
            pltpu.VMEM((D, bpw), jnp.float32),
            pltpu.VMEM((D, bpw), jnp.float32),
            pltpu.SemaphoreType.DMA,
        ],
    )
    def sc_k(ut_hbm, it_hbm, ui_hbm, ii_hbm, uo_hbm, io_hbm,
             uidx_v, iidx_v, buf, uout, iout, sem):
        wid = lax.axis_index("s") * _NC + lax.axis_index("c")
        base = wid * bpw
        iota16 = lax.iota(jnp.int32, 16)
        pltpu.sync_copy(ui_hbm.at[pl.ds(base, bpw)], uidx_v)
        pltpu.sync_copy(ii_hbm.at[pl.ds(base, bpw)], iidx_v)
        nchunks = bpw // 16

        def run_pass(t_hbm, idx_v, out):
            @pl.loop(0, nchunks + 2)
            def _(c):
                @pl.when(c < nchunks)
                def _():
                    c16 = pl.multiple_of(c * 16, 16)
                    vecs = idx_v[pl.ds(c16, 16)]
                    s0 = lax.rem(c, 3) * 16
                    for j in range(16):
                        o = pl.multiple_of((vecs[j] >> 7) * _LANES, _LANES)
                        pltpu.async_copy(t_hbm.at[:, pl.ds(o, _LANES)],
                                         buf.at[s0 + j], sem)

                @pl.when(c > 1)
                def _():
                    cm16 = pl.multiple_of((c - 2) * 16, 16)
                    vecs = idx_v[pl.ds(cm16, 16)]
                    lanes = vecs & (_LANES - 1)
                    s0 = lax.rem(c - 2, 3) * 16
                    for j in range(16):
                        pltpu.make_async_copy(
                            t_hbm.at[:, pl.ds(0, _LANES)],
                            buf.at[s0 + j], sem).wait()
                        s16 = jnp.zeros((16,), jnp.int32) + (s0 + j)
                        g16 = jnp.zeros((16,), jnp.int32) + (cm16 + j)
                        lv = jnp.zeros((16,), jnp.int32) + lanes[j]
                        vec = plsc.load_gather(buf, [s16, iota16, lv])
                        plsc.store_scatter(out, [iota16, g16], vec)

        run_pass(ut_hbm, uidx_v, uout)
        run_pass(it_hbm, iidx_v, iout)
        pltpu.sync_copy(uout, uo_hbm.at[:, pl.ds(base, bpw)])
        pltpu.sync_copy(iout, io_hbm.at[:, pl.ds(base, bpw)])

    return sc_k(user_t, item_t, user_idx, item_idx)


def _mlp_body(xu_ref, xi_ref, w1u_ref, w1i_ref, b1_ref,
              w2_ref, b2_ref, w3_ref, b3_ref, wf_ref, bf_ref, o_ref):
    hp = jax.lax.Precision.HIGHEST
    h = jnp.dot(w1u_ref[...], xu_ref[...], precision=hp)
    h += jnp.dot(w1i_ref[...], xi_ref[...], precision=hp)
    h = jnp.maximum(h + b1_ref[...], 0.0)
    h = jnp.maximum(jnp.dot(w2_ref[...], h, precision=hp) + b2_ref[...], 0.0)
    h = jnp.maximum(jnp.dot(w3_ref[...], h, precision=hp) + b3_ref[...], 0.0)
    logits = jnp.dot(wf_ref[...], h, precision=hp) + bf_ref[...]
    o_ref[...] = jax.nn.sigmoid(logits)


def _tc_mlp_t(xu, xi, W1, b1, W2, b2, W3, b3, Wf, bf):
    D, B = xu.shape
    blk = 4096
    w1ut = W1[:D].T
    w1it = W1[D:].T
    w2t, w3t, wft = W2.T, W3.T, Wf.T
    b1c, b2c, b3c = b1.reshape(-1, 1), b2.reshape(-1, 1), b3.reshape(-1, 1)
    bfc = bf.reshape(1, 1)

    full = lambda shape: pl.BlockSpec(shape, lambda b: (0, 0))
    out = pl.pallas_call(
        _mlp_body,
        grid=(B // blk,),
        in_specs=[
            pl.BlockSpec((D, blk), lambda b: (0, b)),
            pl.BlockSpec((D, blk), lambda b: (0, b)),
            full(w1ut.shape), full(w1it.shape), full(b1c.shape),
            full(w2t.shape), full(b2c.shape),
            full(w3t.shape), full(b3c.shape),
            full(wft.shape), full(bfc.shape),
        ],
        out_specs=pl.BlockSpec((1, blk), lambda b: (0, b)),
        out_shape=jax.ShapeDtypeStruct((1, B), jnp.float32),
        compiler_params=pltpu.CompilerParams(
            dimension_semantics=("parallel",)),
    )(xu, xi, w1ut, w1it, b1c, w2t, b2c, w3t, b3c, wft, bfc)
    return out.reshape(B, 1)


def kernel(user_input, item_input, user_table, item_table,
           W1, b1, W2, b2, W3, b3, Wf, bf):
    xu, xi = _sc_gather_t(user_table.T, item_table.T, user_input, item_input)
    return _tc_mlp_t(xu, xi, W1, b1, W2, b2, W3, b3, Wf, bf)


# speedup vs baseline: 1.0074x; 1.0074x over previous
import dataclasses
import functools

import jax
import jax.numpy as jnp
from jax import lax
from jax.experimental import pallas as pl
from jax.experimental.pallas import tpu as pltpu
from jax.experimental.pallas import tpu_sc as plsc

_NC = 2
_NS = 16
_NW = _NC * _NS
_NBUF = 8
_LANES = 128


def _sc_gather_t(user_t, item_t, user_idx, item_idx):
    D, V = user_t.shape
    B = user_idx.shape[0]
    bpw = B // _NW

    mesh = plsc.VectorSubcoreMesh(core_axis_name="c", subcore_axis_name="s")

    cp = pltpu.CompilerParams()
    if "needs_layout_passes" in pltpu.CompilerParams.__dataclass_fields__:
        cp = dataclasses.replace(cp, needs_layout_passes=False)

    @functools.partial(
        pl.kernel,
        mesh=mesh,
        compiler_params=cp,
        out_type=(jax.ShapeDtypeStruct((D, B), jnp.float32),
                  jax.ShapeDtypeStruct((D, B), jnp.float32)),
        scratch_types=[
            pltpu.VMEM((bpw,), jnp.int32),
            pltpu.VMEM((bpw,), jnp.int32),
            pltpu.VMEM((32, D, _LANES), jnp.float32),
            pltpu.VMEM((D, bpw), jnp.float32),
            pltpu.VMEM((D, bpw), jnp.float32),
            pltpu.SemaphoreType.DMA,
        ],
    )
    def sc_k(ut_hbm, it_hbm, ui_hbm, ii_hbm, uo_hbm, io_hbm,
             uidx_v, iidx_v, buf, uout, iout, sem):
        wid = lax.axis_index("s") * _NC + lax.axis_index("c")
        base = wid * bpw
        iota16 = lax.iota(jnp.int32, 16)
        pltpu.sync_copy(ui_hbm.at[pl.ds(base, bpw)], uidx_v)
        pltpu.sync_copy(ii_hbm.at[pl.ds(base, bpw)], iidx_v)
        nchunks = bpw // 16

        def run_pass(t_hbm, idx_v, out):
            @pl.loop(0, nchunks + 1)
            def _(c):
                @pl.when(c < nchunks)
                def _():
                    c16 = pl.multiple_of(c * 16, 16)
                    vecs = idx_v[pl.ds(c16, 16)]
                    s0 = lax.rem(c, 2) * 16
                    for j in range(16):
                        o = pl.multiple_of((vecs[j] >> 7) * _LANES, _LANES)
                        pltpu.async_copy(t_hbm.at[:, pl.ds(o, _LANES)],
                                         buf.at[s0 + j], sem)

                @pl.when(c > 0)
                def _():
                    cm16 = pl.multiple_of((c - 1) * 16, 16)
                    vecs = idx_v[pl.ds(cm16, 16)]
                    lanes = vecs & (_LANES - 1)
                    s0 = lax.rem(c - 1, 2) * 16
                    for j in range(16):
                        pltpu.make_async_copy(
                            t_hbm.at[:, pl.ds(0, _LANES)],
                            buf.at[s0 + j], sem).wait()
                        s16 = jnp.zeros((16,), jnp.int32) + (s0 + j)
                        g16 = jnp.zeros((16,), jnp.int32) + (cm16 + j)
                        lv = jnp.zeros((16,), jnp.int32) + lanes[j]
                        vec = plsc.load_gather(buf, [s16, iota16, lv])
                        plsc.store_scatter(out, [iota16, g16], vec)

        run_pass(ut_hbm, uidx_v, uout)
        run_pass(it_hbm, iidx_v, iout)
        pltpu.sync_copy(uout, uo_hbm.at[:, pl.ds(base, bpw)])
        pltpu.sync_copy(iout, io_hbm.at[:, pl.ds(base, bpw)])

    return sc_k(user_t, item_t, user_idx, item_idx)


def _dot_t(w_ref, x, hp):
    return lax.dot_general(w_ref[...], x, (((0,), (0,)), ((), ())),
                           precision=hp)


def _mlp_body(xu_ref, xi_ref, w1u_ref, w1i_ref, b1_ref,
              w2_ref, b2_ref, w3_ref, b3_ref, wf_ref, bf_ref, o_ref):
    hp = jax.lax.Precision.HIGHEST
    h = _dot_t(w1u_ref, xu_ref[...], hp) + _dot_t(w1i_ref, xi_ref[...], hp)
    h = jnp.maximum(h + b1_ref[...], 0.0)
    h = jnp.maximum(_dot_t(w2_ref, h, hp) + b2_ref[...], 0.0)
    h = jnp.maximum(_dot_t(w3_ref, h, hp) + b3_ref[...], 0.0)
    logits = _dot_t(wf_ref, h, hp) + bf_ref[...]
    o_ref[...] = jax.nn.sigmoid(logits)


def _tc_mlp_t(xu, xi, W1, b1, W2, b2, W3, b3, Wf, bf):
    D, B = xu.shape
    blk = 2048
    w1u = W1[:D]
    w1i = W1[D:]
    b1c, b2c, b3c = b1.reshape(-1, 1), b2.reshape(-1, 1), b3.reshape(-1, 1)
    bfc = bf.reshape(1, 1)

    full = lambda shape: pl.BlockSpec(shape, lambda b: (0, 0))
    out = pl.pallas_call(
        _mlp_body,
        grid=(B // blk,),
        in_specs=[
            pl.BlockSpec((D, blk), lambda b: (0, b)),
            pl.BlockSpec((D, blk), lambda b: (0, b)),
            full(w1u.shape), full(w1i.shape), full(b1c.shape),
            full(W2.shape), full(b2c.shape),
            full(W3.shape), full(b3c.shape),
            full(Wf.shape), full(bfc.shape),
        ],
        out_specs=pl.BlockSpec((1, blk), lambda b: (0, b)),
        out_shape=jax.ShapeDtypeStruct((1, B), jnp.float32),
        compiler_params=pltpu.CompilerParams(
            dimension_semantics=("parallel",)),
    )(xu, xi, w1u, w1i, b1c, W2, b2c, W3, b3c, Wf, bfc)
    return out.reshape(B, 1)


def kernel(user_input, item_input, user_table, item_table,
           W1, b1, W2, b2, W3, b3, Wf, bf):
    xu, xi = _sc_gather_t(user_table.T, item_table.T, user_input, item_input)
    return _tc_mlp_t(xu, xi, W1, b1, W2, b2, W3, b3, Wf, bf)
